# baseline (speedup 1.0000x reference)
import functools

import jax
import jax.numpy as jnp
from jax import lax
from jax.experimental import pallas as pl
from jax.experimental.pallas import tpu as pltpu
from jax.experimental.pallas import tpu_sc as plsc

_NUM_CORES = 2
_NUM_SUBCORES = 16
_NUM_WORKERS = _NUM_CORES * _NUM_SUBCORES
_CHUNK = 128


def _sc_gather(idx, table, B, D):
    n_chunks = (B // _NUM_WORKERS) // _CHUNK
    b_per_w = n_chunks * _CHUNK
    mesh = plsc.VectorSubcoreMesh(core_axis_name="c", subcore_axis_name="s")

    @functools.partial(
        pl.kernel,
        out_type=jax.ShapeDtypeStruct((B, D), jnp.float32),
        mesh=mesh,
        scratch_types=[
            pltpu.VMEM((n_chunks, _CHUNK), jnp.int32),
            pltpu.VMEM((b_per_w, D), jnp.float32),
            pltpu.SemaphoreType.DMA,
        ],
        compiler_params=pltpu.CompilerParams(use_tc_tiling_on_sc=False),
    )
    def gather_k(idx_hbm, table_hbm, out_hbm, idx_v, rows_v, sem):
        wid = lax.axis_index("s") * _NUM_CORES + lax.axis_index("c")
        base = wid * b_per_w
        pltpu.sync_copy(idx_hbm.at[wid], idx_v)
        copies = [
            pltpu.async_copy(
                table_hbm.at[idx_v.at[j]],
                rows_v.at[pl.ds(j * _CHUNK, _CHUNK)],
                sem,
            )
            for j in range(n_chunks)
        ]
        for c in copies:
            c.wait()
        pltpu.sync_copy(rows_v, out_hbm.at[pl.ds(base, b_per_w)])

    return gather_k(idx, table)


def _mlp(rows, W1, b1, W2, b2, B, D, H):
    BM = 2048

    def mlp_body(h_ref, w1_ref, b1_ref, w2_ref, b2_ref, o_ref):
        h = h_ref[...]
        z = jnp.dot(h, w1_ref[...], preferred_element_type=jnp.float32)
        z = z + b1_ref[...]
        z = z * jax.nn.sigmoid(z)
        o_ref[...] = (
            jnp.dot(z, w2_ref[...], preferred_element_type=jnp.float32)
            + b2_ref[...]
        )

    return pl.pallas_call(
        mlp_body,
        grid=(B // BM,),
        in_specs=[
            pl.BlockSpec((BM, D), lambda i: (i, 0)),
            pl.BlockSpec((D, H), lambda i: (0, 0)),
            pl.BlockSpec((1, H), lambda i: (0, 0)),
            pl.BlockSpec((H, D), lambda i: (0, 0)),
            pl.BlockSpec((1, D), lambda i: (0, 0)),
        ],
        out_specs=pl.BlockSpec((BM, D), lambda i: (i, 0)),
        out_shape=jax.ShapeDtypeStruct((B, D), jnp.float32),
    )(rows, W1, b1, W2, b2)


def kernel(x, table, W1, b1, W2, b2):
    B, = x.shape
    V, D = table.shape
    H = W1.shape[1]
    idx = x.astype(jnp.int32).reshape(
        _NUM_WORKERS, (B // _NUM_WORKERS) // _CHUNK, _CHUNK
    )
    rows = _sc_gather(idx, table, B, D)
    return _mlp(rows, W1, b1.reshape(1, H), W2, b2.reshape(1, D), B, D, H)
